# baseline (speedup 1.0000x reference)
import functools

import jax
import jax.numpy as jnp
from jax import lax
from jax.experimental import pallas as pl
from jax.experimental.pallas import tpu as pltpu
from jax.experimental.pallas import tpu_sc as plsc

EMB_DIM = 32
LO, HI = 0.8, 5.2
LANES = 16


def _build_sc_kernel(batch, emb):
    info = plsc.get_sparse_core_info()
    nw = info.num_cores * info.num_subcores
    nc = info.num_cores
    b_per_w = batch // nw
    chunks = b_per_w // LANES
    mesh = plsc.VectorSubcoreMesh(core_axis_name="c", subcore_axis_name="s")

    @functools.partial(
        pl.kernel,
        out_type=jax.ShapeDtypeStruct((batch,), jnp.float32),
        mesh=mesh,
        scratch_types=[
            pltpu.VMEM((b_per_w,), jnp.int32),
            pltpu.VMEM((b_per_w,), jnp.int32),
            pltpu.VMEM((b_per_w, emb), jnp.float32),
            pltpu.VMEM((b_per_w, emb), jnp.float32),
            pltpu.VMEM((b_per_w,), jnp.float32),
            pltpu.VMEM((b_per_w,), jnp.float32),
            pltpu.VMEM((b_per_w,), jnp.float32),
            pltpu.SemaphoreType.DMA,
            pltpu.SemaphoreType.DMA,
            pltpu.SemaphoreType.DMA,
            pltpu.SemaphoreType.DMA,
        ],
        compiler_params=pltpu.CompilerParams(
            needs_layout_passes=False, use_tc_tiling_on_sc=False),
    )
    def sc_kernel(uidx_hbm, midx_hbm, uemb_hbm, iemb_hbm, ubias_hbm, ibias_hbm,
                  out_hbm,
                  uidx_v, midx_v, urows_v, irows_v, ubias_v, ibias_v, out_v,
                  sem_u, sem_i, sem_ub, sem_ib):
        wid = lax.axis_index("s") * nc + lax.axis_index("c")
        base = wid * b_per_w
        pltpu.sync_copy(uidx_hbm.at[pl.ds(base, b_per_w)], uidx_v)
        pltpu.sync_copy(midx_hbm.at[pl.ds(base, b_per_w)], midx_v)
        cu = pltpu.async_copy(uemb_hbm.at[uidx_v], urows_v, sem_u)
        ci = pltpu.async_copy(iemb_hbm.at[midx_v], irows_v, sem_i)
        cub = pltpu.async_copy(ubias_hbm.at[uidx_v], ubias_v, sem_ub)
        cib = pltpu.async_copy(ibias_hbm.at[midx_v], ibias_v, sem_ib)
        cu.wait()
        ci.wait()
        cub.wait()
        cib.wait()

        iot = lax.iota(jnp.int32, 16)

        def chunk_body(c, carry):
            rowids = c * LANES + iot
            accs = [jnp.zeros((LANES,), jnp.float32) for _ in range(4)]
            for e in range(emb):
                col = jnp.full((LANES,), e, jnp.int32)
                uu = plsc.load_gather(urows_v, [rowids, col])
                vv = plsc.load_gather(irows_v, [rowids, col])
                accs[e % 4] = accs[e % 4] + uu * vv
            x = (accs[0] + accs[1]) + (accs[2] + accs[3])
            x = x + ubias_v[pl.ds(c * LANES, LANES)]
            x = x + ibias_v[pl.ds(c * LANES, LANES)]
            sig = 1.0 / (1.0 + jnp.exp(-x))
            out_v[pl.ds(c * LANES, LANES)] = sig * (HI - LO) + LO
            return carry

        lax.fori_loop(0, chunks, chunk_body, 0)
        pltpu.sync_copy(out_v, out_hbm.at[pl.ds(base, b_per_w)])

    return sc_kernel


def kernel(user_idx, movie_idx, user_emb_table, item_emb_table,
           user_bias_table, item_bias_table):
    batch = user_idx.shape[0]
    emb = user_emb_table.shape[1]
    sc = _build_sc_kernel(batch, emb)
    return sc(
        user_idx.astype(jnp.int32),
        movie_idx.astype(jnp.int32),
        user_emb_table,
        item_emb_table,
        user_bias_table.reshape(-1),
        item_bias_table.reshape(-1),
    )
